# baseline (speedup 1.0000x reference)
import functools

import jax
import jax.numpy as jnp
from jax import lax
from jax.experimental import pallas as pl
from jax.experimental.pallas import tpu as pltpu
from jax.experimental.pallas import tpu_sc as plsc

N = 10000
NP = 10240
E = 160000
EPAD = 163840
B = 64
ERWS = EPAD // B
NF = 9
VOCAB = 128
VROWS = NF * VOCAB
F = N * NF
FPAD = 98304
FRWS = FPAD // B
D = 256
HD = 128
STRIPE = NP // 16
GRID = 16

@functools.cache
def _get_mesh():
    return plsc.VectorSubcoreMesh(core_axis_name="c", subcore_axis_name="s")


@functools.cache
def _make_agg(trows, irws, hd=HD):
    rpt = irws // 16
    nch = rpt // 16

    @functools.partial(
        pl.kernel,
        out_type=jax.ShapeDtypeStruct((2 * NP, hd), jnp.float32),
        mesh=_get_mesh(),
        scratch_types=[
            pltpu.VMEM_SHARED((NP, hd), jnp.float32),
            pltpu.VMEM((16, B), jnp.int32),
            pltpu.VMEM((16, B), jnp.int32),
            pltpu.VMEM((B, hd), jnp.float32),
            pltpu.VMEM((B, hd), jnp.float32),
            pltpu.VMEM((B, hd), jnp.float32),
            pltpu.VMEM((B, hd), jnp.float32),
            pltpu.SemaphoreType.DMA, pltpu.SemaphoreType.DMA,
            pltpu.SemaphoreType.DMA, pltpu.SemaphoreType.DMA,
            pltpu.SemaphoreType.DMA, pltpu.SemaphoreType.DMA,
            pltpu.SemaphoreType.DMA, pltpu.SemaphoreType.DMA,
        ],
    )
    def k(table, g01, si, zf, z_o, acc, vgi, vsi, rb0, rb1, rb2, rb3,
          gs0, gs1, gs2, gs3, ss0, ss1, ss2, ss3):
        bufs = (rb0, rb1, rb2, rb3)
        gsem = (gs0, gs1, gs2, gs3)
        ssem = (ss0, ss1, ss2, ss3)
        c = lax.axis_index("c")
        s = lax.axis_index("s")
        st = pl.ds(s * STRIPE, STRIPE)
        pltpu.sync_copy(zf, acc.at[st])
        plsc.subcore_barrier()

        def chunk(kk, carry):
            row0 = s * rpt + kk * 16
            pltpu.sync_copy(g01.at[pl.ds(c * irws + row0, 16)], vgi)
            pltpu.sync_copy(si.at[pl.ds(row0, 16)], vsi)
            for b in range(4):
                pltpu.async_copy(table.at[vgi.at[b]], bufs[b], gsem[b])
            for i0 in (0, 4, 8, 12):
                for b in range(4):
                    pltpu.make_async_copy(table.at[vgi.at[i0 + b]], bufs[b],
                                          gsem[b]).wait()
                    pltpu.async_copy(bufs[b], acc.at[vsi.at[i0 + b]],
                                     ssem[b], add=True)
                if i0 < 12:
                    for b in range(4):
                        pltpu.make_async_copy(bufs[b], acc.at[vsi.at[i0 + b]],
                                              ssem[b]).wait()
                        pltpu.async_copy(table.at[vgi.at[i0 + 4 + b]],
                                         bufs[b], gsem[b])
            for b in range(4):
                pltpu.make_async_copy(bufs[b], acc.at[vsi.at[12 + b]],
                                      ssem[b]).wait()
            return carry

        lax.fori_loop(0, nch, chunk, 0)
        plsc.subcore_barrier()
        pltpu.sync_copy(acc.at[st], z_o.at[pl.ds(c * NP + s * STRIPE, STRIPE)])

    return k


def _sc_emb(*args):
    return _make_agg(2 * VROWS, FRWS)(*args)


def _sc_edge(*args):
    return _make_agg(2 * NP, ERWS)(*args)


@functools.cache
def _make_deg():
    rpt = ERWS // 16

    @functools.partial(
        pl.kernel,
        out_type=jax.ShapeDtypeStruct((2 * NP, HD), jnp.float32),
        mesh=_get_mesh(),
        scratch_types=[
            pltpu.VMEM_SHARED((NP, HD), jnp.float32),
            pltpu.VMEM((ERWS // 16, B), jnp.int32),
            pltpu.VMEM((B, HD), jnp.float32),
            pltpu.SemaphoreType.DMA,
        ],
    )
    def k(ones_hbm, si, zf, z_o, acc, vsi, vones, dsem):
        c = lax.axis_index("c")
        s = lax.axis_index("s")
        st = pl.ds(s * STRIPE, STRIPE)
        pltpu.sync_copy(zf, acc.at[st])
        pltpu.sync_copy(ones_hbm, vones)
        pltpu.sync_copy(si.at[pl.ds(c * ERWS + s * rpt, rpt)], vsi)
        plsc.subcore_barrier()

        def group(g, carry):
            i0 = 8 * g
            for b in range(8):
                pltpu.async_copy(vones, acc.at[vsi.at[i0 + b]], dsem,
                                 add=True)
            for b in range(8):
                pltpu.make_async_copy(vones, acc.at[vsi.at[i0 + b]],
                                      dsem).wait()
            return carry

        lax.fori_loop(0, rpt // 8, group, 0)
        plsc.subcore_barrier()
        pltpu.sync_copy(acc.at[st], z_o.at[pl.ds(c * NP + s * STRIPE, STRIPE)])

    return k


def _sc_deg(*args):
    return _make_deg()(*args)


def _norm(deg_blk):
    d = deg_blk[0][:, 0:1]
    return jnp.where(d > 0.0, lax.rsqrt(d), 0.0)


def _tc_scale_body(xlo, xhi, degs, y_ref):
    ns = _norm(degs)
    y_ref[:, 0, :] = xlo[0] * ns
    y_ref[:, 1, :] = xhi[0] * ns


def _tc_scale(xemb2, deg2):
    return pl.pallas_call(
        _tc_scale_body,
        grid=(GRID,),
        in_specs=[
            pl.BlockSpec((1, STRIPE, HD), lambda i: (0, i, 0)),
            pl.BlockSpec((1, STRIPE, HD), lambda i: (1, i, 0)),
            pl.BlockSpec((1, STRIPE, HD), lambda i: (0, i, 0)),
        ],
        out_specs=pl.BlockSpec((STRIPE, 2, HD), lambda i: (i, 0, 0)),
        out_shape=jax.ShapeDtypeStruct((NP, 2, HD), jnp.float32),
    )(xemb2, xemb2, deg2)


def _tc_layer_body(zlo, zhi, degs, degd, w, b, y_ref):
    nd = _norm(degd)
    ns = _norm(degs)
    zz = jnp.concatenate([zlo[0], zhi[0]], axis=1) * nd
    x = jnp.maximum(jnp.dot(zz, w[...],
                            preferred_element_type=jnp.float32) + b[...], 0.0)
    x = x * ns
    y_ref[:, 0, :] = x[:, :HD]
    y_ref[:, 1, :] = x[:, HD:]


def _tc_layer(z2, deg2, w, b):
    return pl.pallas_call(
        _tc_layer_body,
        grid=(GRID,),
        in_specs=[
            pl.BlockSpec((1, STRIPE, HD), lambda i: (0, i, 0)),
            pl.BlockSpec((1, STRIPE, HD), lambda i: (1, i, 0)),
            pl.BlockSpec((1, STRIPE, HD), lambda i: (0, i, 0)),
            pl.BlockSpec((1, STRIPE, HD), lambda i: (1, i, 0)),
            pl.BlockSpec((D, D), lambda i: (0, 0)),
            pl.BlockSpec((1, D), lambda i: (0, 0)),
        ],
        out_specs=pl.BlockSpec((STRIPE, 2, HD), lambda i: (i, 0, 0)),
        out_shape=jax.ShapeDtypeStruct((NP, 2, HD), jnp.float32),
    )(z2, z2, deg2, deg2, w, b)


def _tc_final_body(zlo, zhi, degd, w, b, wr, br, out_ref, acc):
    i = pl.program_id(0)

    @pl.when(i == 0)
    def _():
        acc[...] = jnp.zeros_like(acc)

    nd = _norm(degd)
    zz = jnp.concatenate([zlo[0], zhi[0]], axis=1) * nd
    x = jnp.maximum(jnp.dot(zz, w[...],
                            preferred_element_type=jnp.float32) + b[...], 0.0)
    rows = lax.broadcasted_iota(jnp.int32, (STRIPE, D), 0) + i * STRIPE
    x = jnp.where(rows < N, x, 0.0)
    acc[...] += jnp.sum(x, axis=0, keepdims=True)

    @pl.when(i == GRID - 1)
    def _():
        out_ref[...] = jnp.dot(acc[...] * (1.0 / N), wr[...],
                               preferred_element_type=jnp.float32) + br[...]


def _tc_final(z2, deg2, w, b, wr, br):
    return pl.pallas_call(
        _tc_final_body,
        grid=(GRID,),
        in_specs=[
            pl.BlockSpec((1, STRIPE, HD), lambda i: (0, i, 0)),
            pl.BlockSpec((1, STRIPE, HD), lambda i: (1, i, 0)),
            pl.BlockSpec((1, STRIPE, HD), lambda i: (1, i, 0)),
            pl.BlockSpec((D, D), lambda i: (0, 0)),
            pl.BlockSpec((1, D), lambda i: (0, 0)),
            pl.BlockSpec((D, 1), lambda i: (0, 0)),
            pl.BlockSpec((1, 1), lambda i: (0, 0)),
        ],
        out_specs=pl.BlockSpec((1, 1), lambda i: (0, 0)),
        out_shape=jax.ShapeDtypeStruct((1, 1), jnp.float32),
        scratch_shapes=[pltpu.VMEM((1, D), jnp.float32)],
    )(z2, z2, deg2, w, b, wr, br)


def kernel(h, edge_index, atom_emb, W1, b1, W2, b2, Wr, br):
    i32 = jnp.int32
    f32 = jnp.float32
    src = edge_index[0]
    dst = edge_index[1]

    pe = EPAD - E
    gpad = jnp.arange(pe, dtype=i32) % N
    spad = N + jnp.arange(pe, dtype=i32) % (NP - N)
    src_p = jnp.concatenate([src, gpad])
    g01 = jnp.concatenate([(2 * src_p).reshape(ERWS, B),
                           (2 * src_p + 1).reshape(ERWS, B)])
    sdst = jnp.concatenate([dst, spad]).reshape(ERWS, B)
    ssrcd = jnp.concatenate([src, spad]).reshape(ERWS, B)

    pf = FPAD - F
    flat = (h.astype(i32) + (jnp.arange(NF, dtype=i32) * VOCAB)[None, :])
    gh0 = jnp.concatenate([flat.reshape(-1),
                           jnp.arange(pf, dtype=i32) % VROWS]).reshape(FRWS, B)
    gh01 = jnp.concatenate([gh0, gh0 + VROWS])
    snode = jnp.concatenate([
        jnp.repeat(jnp.arange(N, dtype=i32), NF),
        N + jnp.arange(pf, dtype=i32) % (NP - N)]).reshape(FRWS, B)

    emb2 = atom_emb.reshape(VROWS, D)
    emb_cm = jnp.concatenate([emb2[:, :HD], emb2[:, HD:]], axis=0)
    zf = jnp.zeros((STRIPE, HD), f32)
    ones_b = jnp.ones((B, HD), f32)
    sdual = jnp.concatenate([ssrcd, sdst])

    deg2 = _sc_deg(ones_b, sdual, zf).reshape(2, NP, HD)
    xemb = _sc_emb(emb_cm, gh01, snode, zf)
    y0 = _tc_scale(xemb.reshape(2, NP, HD), deg2)
    z1 = _sc_edge(y0.reshape(2 * NP, HD), g01, sdst, zf)
    y1 = _tc_layer(z1.reshape(2, NP, HD), deg2, W1, b1.reshape(1, D))
    z2 = _sc_edge(y1.reshape(2 * NP, HD), g01, sdst, zf)
    return _tc_final(z2.reshape(2, NP, HD), deg2, W2, b2.reshape(1, D),
                     Wr, br.reshape(1, 1))


# speedup vs baseline: 1.0783x; 1.0783x over previous
import functools

import jax
import jax.numpy as jnp
from jax import lax
from jax.experimental import pallas as pl
from jax.experimental.pallas import tpu as pltpu
from jax.experimental.pallas import tpu_sc as plsc

N = 10000
NP = 10240
E = 160000
EPAD = 163840
B = 128
ERWS = EPAD // B
NF = 9
VOCAB = 128
VROWS = NF * VOCAB
F = N * NF
FPAD = 98304
FRWS = FPAD // B
D = 256
HD = 128
STRIPE = NP // 16
GRID = 16

@functools.cache
def _get_mesh():
    return plsc.VectorSubcoreMesh(core_axis_name="c", subcore_axis_name="s")


@functools.cache
def _make_agg(trows, irws, hd=HD):
    rpt = irws // 16
    nch = rpt // 16

    @functools.partial(
        pl.kernel,
        out_type=jax.ShapeDtypeStruct((2 * NP, hd), jnp.float32),
        mesh=_get_mesh(),
        scratch_types=[
            pltpu.VMEM_SHARED((NP, hd), jnp.float32),
            pltpu.VMEM((16, B), jnp.int32),
            pltpu.VMEM((16, B), jnp.int32),
            pltpu.VMEM((B, hd), jnp.float32),
            pltpu.VMEM((B, hd), jnp.float32),
            pltpu.SemaphoreType.DMA,
            pltpu.SemaphoreType.DMA,
        ],
    )
    def k(table, g01, si, zf, z_o, acc, vgi, vsi, vra, vrb, sem_a, sem_b):
        c = lax.axis_index("c")
        s = lax.axis_index("s")
        st = pl.ds(s * STRIPE, STRIPE)
        pltpu.sync_copy(zf, acc.at[st])
        plsc.subcore_barrier()

        def chunk(kk, carry):
            row0 = s * rpt + kk * 16
            pltpu.sync_copy(g01.at[pl.ds(c * irws + row0, 16)], vgi)
            pltpu.sync_copy(si.at[pl.ds(row0, 16)], vsi)
            pltpu.async_copy(table.at[vgi.at[0]], vra, sem_a)

            def pair(i, carry2):
                b0 = 2 * i
                pltpu.async_copy(table.at[vgi.at[b0 + 1]], vrb, sem_b)
                pltpu.make_async_copy(table.at[vgi.at[b0]], vra, sem_a).wait()
                pltpu.sync_copy(vra, acc.at[vsi.at[b0]], add=True)

                @pl.when(i < 8 - 1)
                def _():
                    pltpu.async_copy(table.at[vgi.at[b0 + 2]], vra, sem_a)

                pltpu.make_async_copy(table.at[vgi.at[b0 + 1]], vrb,
                                      sem_b).wait()
                pltpu.sync_copy(vrb, acc.at[vsi.at[b0 + 1]], add=True)
                return carry2

            lax.fori_loop(0, 8, pair, 0)
            return carry

        lax.fori_loop(0, nch, chunk, 0)
        plsc.subcore_barrier()
        pltpu.sync_copy(acc.at[st], z_o.at[pl.ds(c * NP + s * STRIPE, STRIPE)])

    return k


def _sc_emb(*args):
    return _make_agg(2 * VROWS, FRWS)(*args)


def _sc_edge(*args):
    return _make_agg(2 * NP, ERWS)(*args)


@functools.cache
def _make_deg():
    rpt = ERWS // 16

    @functools.partial(
        pl.kernel,
        out_type=jax.ShapeDtypeStruct((2 * NP, HD), jnp.float32),
        mesh=_get_mesh(),
        scratch_types=[
            pltpu.VMEM_SHARED((NP, HD), jnp.float32),
            pltpu.VMEM((ERWS // 16, B), jnp.int32),
            pltpu.VMEM((B, HD), jnp.float32),
            pltpu.SemaphoreType.DMA,
        ],
    )
    def k(ones_hbm, si, zf, z_o, acc, vsi, vones, dsem):
        c = lax.axis_index("c")
        s = lax.axis_index("s")
        st = pl.ds(s * STRIPE, STRIPE)
        pltpu.sync_copy(zf, acc.at[st])
        pltpu.sync_copy(ones_hbm, vones)
        pltpu.sync_copy(si.at[pl.ds(c * ERWS + s * rpt, rpt)], vsi)
        plsc.subcore_barrier()

        def group(g, carry):
            i0 = 8 * g
            for b in range(8):
                pltpu.async_copy(vones, acc.at[vsi.at[i0 + b]], dsem,
                                 add=True)
            for b in range(8):
                pltpu.make_async_copy(vones, acc.at[vsi.at[i0 + b]],
                                      dsem).wait()
            return carry

        lax.fori_loop(0, rpt // 8, group, 0)
        plsc.subcore_barrier()
        pltpu.sync_copy(acc.at[st], z_o.at[pl.ds(c * NP + s * STRIPE, STRIPE)])

    return k


def _sc_deg(*args):
    return _make_deg()(*args)


def _norm(deg_blk):
    d = deg_blk[0][:, 0:1]
    return jnp.where(d > 0.0, lax.rsqrt(d), 0.0)


def _tc_scale_body(xlo, xhi, degs, y_ref):
    ns = _norm(degs)
    y_ref[:, 0, :] = xlo[0] * ns
    y_ref[:, 1, :] = xhi[0] * ns


def _tc_scale(xemb2, deg2):
    return pl.pallas_call(
        _tc_scale_body,
        grid=(GRID,),
        in_specs=[
            pl.BlockSpec((1, STRIPE, HD), lambda i: (0, i, 0)),
            pl.BlockSpec((1, STRIPE, HD), lambda i: (1, i, 0)),
            pl.BlockSpec((1, STRIPE, HD), lambda i: (0, i, 0)),
        ],
        out_specs=pl.BlockSpec((STRIPE, 2, HD), lambda i: (i, 0, 0)),
        out_shape=jax.ShapeDtypeStruct((NP, 2, HD), jnp.float32),
    )(xemb2, xemb2, deg2)


def _tc_layer_body(zlo, zhi, degs, degd, w, b, y_ref):
    nd = _norm(degd)
    ns = _norm(degs)
    zz = jnp.concatenate([zlo[0], zhi[0]], axis=1) * nd
    x = jnp.maximum(jnp.dot(zz, w[...],
                            preferred_element_type=jnp.float32) + b[...], 0.0)
    x = x * ns
    y_ref[:, 0, :] = x[:, :HD]
    y_ref[:, 1, :] = x[:, HD:]


def _tc_layer(z2, deg2, w, b):
    return pl.pallas_call(
        _tc_layer_body,
        grid=(GRID,),
        in_specs=[
            pl.BlockSpec((1, STRIPE, HD), lambda i: (0, i, 0)),
            pl.BlockSpec((1, STRIPE, HD), lambda i: (1, i, 0)),
            pl.BlockSpec((1, STRIPE, HD), lambda i: (0, i, 0)),
            pl.BlockSpec((1, STRIPE, HD), lambda i: (1, i, 0)),
            pl.BlockSpec((D, D), lambda i: (0, 0)),
            pl.BlockSpec((1, D), lambda i: (0, 0)),
        ],
        out_specs=pl.BlockSpec((STRIPE, 2, HD), lambda i: (i, 0, 0)),
        out_shape=jax.ShapeDtypeStruct((NP, 2, HD), jnp.float32),
    )(z2, z2, deg2, deg2, w, b)


def _tc_final_body(zlo, zhi, degd, w, b, wr, br, out_ref, acc):
    i = pl.program_id(0)

    @pl.when(i == 0)
    def _():
        acc[...] = jnp.zeros_like(acc)

    nd = _norm(degd)
    zz = jnp.concatenate([zlo[0], zhi[0]], axis=1) * nd
    x = jnp.maximum(jnp.dot(zz, w[...],
                            preferred_element_type=jnp.float32) + b[...], 0.0)
    rows = lax.broadcasted_iota(jnp.int32, (STRIPE, D), 0) + i * STRIPE
    x = jnp.where(rows < N, x, 0.0)
    acc[...] += jnp.sum(x, axis=0, keepdims=True)

    @pl.when(i == GRID - 1)
    def _():
        out_ref[...] = jnp.dot(acc[...] * (1.0 / N), wr[...],
                               preferred_element_type=jnp.float32) + br[...]


def _tc_final(z2, deg2, w, b, wr, br):
    return pl.pallas_call(
        _tc_final_body,
        grid=(GRID,),
        in_specs=[
            pl.BlockSpec((1, STRIPE, HD), lambda i: (0, i, 0)),
            pl.BlockSpec((1, STRIPE, HD), lambda i: (1, i, 0)),
            pl.BlockSpec((1, STRIPE, HD), lambda i: (1, i, 0)),
            pl.BlockSpec((D, D), lambda i: (0, 0)),
            pl.BlockSpec((1, D), lambda i: (0, 0)),
            pl.BlockSpec((D, 1), lambda i: (0, 0)),
            pl.BlockSpec((1, 1), lambda i: (0, 0)),
        ],
        out_specs=pl.BlockSpec((1, 1), lambda i: (0, 0)),
        out_shape=jax.ShapeDtypeStruct((1, 1), jnp.float32),
        scratch_shapes=[pltpu.VMEM((1, D), jnp.float32)],
    )(z2, z2, deg2, w, b, wr, br)


def kernel(h, edge_index, atom_emb, W1, b1, W2, b2, Wr, br):
    i32 = jnp.int32
    f32 = jnp.float32
    src = edge_index[0]
    dst = edge_index[1]

    pe = EPAD - E
    gpad = jnp.arange(pe, dtype=i32) % N
    spad = N + jnp.arange(pe, dtype=i32) % (NP - N)
    src_p = jnp.concatenate([src, gpad])
    g01 = jnp.concatenate([(2 * src_p).reshape(ERWS, B),
                           (2 * src_p + 1).reshape(ERWS, B)])
    sdst = jnp.concatenate([dst, spad]).reshape(ERWS, B)
    ssrcd = jnp.concatenate([src, spad]).reshape(ERWS, B)

    pf = FPAD - F
    flat = (h.astype(i32) + (jnp.arange(NF, dtype=i32) * VOCAB)[None, :])
    gh0 = jnp.concatenate([flat.reshape(-1),
                           jnp.arange(pf, dtype=i32) % VROWS]).reshape(FRWS, B)
    gh01 = jnp.concatenate([gh0, gh0 + VROWS])
    snode = jnp.concatenate([
        jnp.repeat(jnp.arange(N, dtype=i32), NF),
        N + jnp.arange(pf, dtype=i32) % (NP - N)]).reshape(FRWS, B)

    emb2 = atom_emb.reshape(VROWS, D)
    emb_cm = jnp.concatenate([emb2[:, :HD], emb2[:, HD:]], axis=0)
    zf = jnp.zeros((STRIPE, HD), f32)
    ones_b = jnp.ones((B, HD), f32)
    sdual = jnp.concatenate([ssrcd, sdst])

    deg2 = _sc_deg(ones_b, sdual, zf).reshape(2, NP, HD)
    xemb = _sc_emb(emb_cm, gh01, snode, zf)
    y0 = _tc_scale(xemb.reshape(2, NP, HD), deg2)
    z1 = _sc_edge(y0.reshape(2 * NP, HD), g01, sdst, zf)
    y1 = _tc_layer(z1.reshape(2, NP, HD), deg2, W1, b1.reshape(1, D))
    z2 = _sc_edge(y1.reshape(2 * NP, HD), g01, sdst, zf)
    return _tc_final(z2.reshape(2, NP, HD), deg2, W2, b2.reshape(1, D),
                     Wr, br.reshape(1, 1))
